# baseline (speedup 1.0000x reference)
import functools

import jax
import jax.numpy as jnp
from jax import lax
from jax.experimental import pallas as pl
from jax.experimental.pallas import tpu as pltpu
from jax.experimental.pallas import tpu_sc as plsc

N = 10000
E = 320000
F_IN = 128
F = 256
HALF = 128
NC = 2
NS = 16
CHUNK = 128
EPT = E // NS
NFULL = EPT // CHUNK
REM = EPT - NFULL * CHUNK
ROWS_T = 640
ROWS_LAST = N - (NS - 1) * ROWS_T
DEGW = 16

@functools.cache
def _mesh():
    return plsc.VectorSubcoreMesh(core_axis_name="c", subcore_axis_name="s",
                                  num_cores=NC, num_subcores=NS)


NP = NS * ROWS_T


def _sc_degree(dst1p, ones_stage):
    return pl.kernel(
        _sc_degree_body,
        out_type=jax.ShapeDtypeStruct((NP,), jnp.float32),
        mesh=_mesh(),
        scratch_types=[
            pltpu.VMEM_SHARED((NP,), jnp.float32),
            pltpu.VMEM((ROWS_T,), jnp.float32),
            pltpu.VMEM((CHUNK,), jnp.float32),
            pltpu.VMEM((CHUNK,), jnp.int32),
            pltpu.VMEM((CHUNK,), jnp.int32),
            pltpu.VMEM((CHUNK,), jnp.int32),
            pltpu.VMEM((CHUNK,), jnp.int32),
            pltpu.SemaphoreType.DMA,
            pltpu.SemaphoreType.DMA,
            pltpu.SemaphoreType.DMA,
            pltpu.SemaphoreType.DMA,
        ],
    )(dst1p, ones_stage)


def _sc_degree_body(dst_hbm, ones_hbm, out_hbm, acc, ones_t, ones_c,
                    dst0, dst1, dst2, dst3, s0, s1, s2, s3):
    cid = lax.axis_index("c")
    sid = lax.axis_index("s")
    row0 = sid * ROWS_T
    D = (dst0, dst1, dst2, dst3)
    SEM = (s0, s1, s2, s3)

    pltpu.sync_copy(ones_hbm, ones_t)
    pltpu.sync_copy(ones_hbm.at[pl.ds(0, CHUNK)], ones_c)
    pltpu.sync_copy(ones_t, acc.at[pl.ds(row0, ROWS_T)])

    plsc.subcore_barrier()

    ebase = sid * (ERT * CHUNK)

    def loadD(k, q):
        pltpu.async_copy(dst_hbm.at[pl.ds(ebase + k * CHUNK, CHUNK)],
                         D[q], SEM[q])

    def waitD(q):
        pltpu.make_async_copy(dst_hbm.at[pl.ds(0, CHUNK)],
                              D[q], SEM[q]).wait()

    for q in range(4):
        loadD(q, q)

    def body(p, c):
        for i in range(4):
            waitD(i)
            pltpu.sync_copy(ones_c, acc.at[D[i]], add=True)
            loadD(4 * p + i + 4, i)
        return c

    lax.fori_loop(0, ERT // 4 - 1, body, 0)
    for i in range(4):
        waitD(i)
        pltpu.sync_copy(ones_c, acc.at[D[i]], add=True)

    plsc.subcore_barrier()

    @pl.when(cid == 0)
    def _():
        pltpu.sync_copy(acc.at[pl.ds(row0, ROWS_T)],
                        out_hbm.at[pl.ds(row0, ROWS_T)])


ERT = 160
E_P = ERT * NS * CHUNK
ACC_PAD = 64
ACC_R = N + ACC_PAD


BLK = 8
NBLK = ERT // BLK


def _sc_aggregate(hp, src1p, dst2d):
    return pl.kernel(
        _sc_aggregate_body,
        out_type=jax.ShapeDtypeStruct((NC * N, HALF), jnp.float32),
        mesh=_mesh(),
        scratch_types=[
            pltpu.VMEM_SHARED((ACC_R, HALF), jnp.float32),
            pltpu.VMEM((BLK * CHUNK,), jnp.int32),
            pltpu.VMEM((BLK * CHUNK,), jnp.int32),
            pltpu.VMEM((BLK, CHUNK), jnp.int32),
            pltpu.VMEM((BLK, CHUNK), jnp.int32),
            pltpu.VMEM((CHUNK, HALF), jnp.float32),
            pltpu.VMEM((CHUNK, HALF), jnp.float32),
            pltpu.SemaphoreType.DMA,
            pltpu.SemaphoreType.DMA,
            pltpu.SemaphoreType.DMA,
            pltpu.SemaphoreType.DMA,
        ],
    )(hp, src1p, dst2d)


def _sc_aggregate_body(hp_hbm, src_hbm, dst_hbm, out_hbm, acc,
                       srcb0, srcb1, dstb0, dstb1, rowsA, rowsB,
                       sb0, sb1, sgA, sgB):
    cid = lax.axis_index("c")
    sid = lax.axis_index("s")
    base = cid * N
    row0 = sid * ROWS_T
    SRCB = (srcb0, srcb1)
    DSTB = (dstb0, dstb1)
    SB = (sb0, sb1)
    ROWS = (rowsA, rowsB)
    SG = (sgA, sgB)

    @pl.when(sid < NS - 1)
    def _():
        pltpu.sync_copy(hp_hbm.at[pl.ds(base + row0, ROWS_T)],
                        acc.at[pl.ds(row0, ROWS_T)])

    @pl.when(sid == NS - 1)
    def _():
        pltpu.sync_copy(hp_hbm.at[pl.ds(base + row0, ROWS_LAST)],
                        acc.at[pl.ds(row0, ROWS_LAST)])

    plsc.subcore_barrier()

    ebase = sid * (ERT * CHUNK)
    brow = sid * ERT

    def loadblk(b, q):
        pltpu.async_copy(src_hbm.at[pl.ds(ebase + b * (BLK * CHUNK),
                                          BLK * CHUNK)], SRCB[q], SB[q])
        pltpu.async_copy(dst_hbm.at[pl.ds(brow + b * BLK, BLK)],
                         DSTB[q], SB[q])

    def waitblk(q):
        pltpu.make_async_copy(src_hbm.at[pl.ds(0, BLK * CHUNK)],
                              SRCB[q], SB[q]).wait()
        pltpu.make_async_copy(dst_hbm.at[pl.ds(0, BLK)],
                              DSTB[q], SB[q]).wait()

    def addbase(q):
        for j in range(BLK * CHUNK // 16):
            sl = pl.ds(j * 16, 16)
            SRCB[q][sl] = SRCB[q][sl] + base

    def start_gather(q, i, r):
        pltpu.async_copy(hp_hbm.at[SRCB[q].at[pl.ds(i * CHUNK, CHUNK)]],
                         ROWS[r], SG[r])

    def wait_gather(r):
        pltpu.make_async_copy(hp_hbm.at[SRCB[0].at[pl.ds(0, CHUNK)]],
                              ROWS[r], SG[r]).wait()

    def scatter(q, i, r):
        pltpu.sync_copy(ROWS[r], acc.at[DSTB[q].at[i]], add=True)

    loadblk(0, 0)
    loadblk(1, 1)
    waitblk(0)
    addbase(0)
    start_gather(0, 0, 0)

    def halfblock(q, next_gather, pre_seam=None):
        for i in range(BLK):
            r = i % 2
            wait_gather(r)
            if i == BLK - 1 and pre_seam is not None:
                pre_seam()
            if i < BLK - 1:
                start_gather(q, i + 1, 1 - r)
            elif next_gather is not None:
                start_gather(next_gather, 0, 1 - r)
            scatter(q, i, r)

    def ready(q):
        def f():
            waitblk(q)
            addbase(q)
        return f

    def body(p, c):
        b0 = 2 * p
        halfblock(0, 1, pre_seam=ready(1))
        loadblk(b0 + 2, 0)
        halfblock(1, 0, pre_seam=ready(0))
        loadblk(b0 + 3, 1)
        return c

    lax.fori_loop(0, NBLK // 2 - 1, body, 0)
    halfblock(0, 1, pre_seam=ready(1))
    halfblock(1, None)

    plsc.subcore_barrier()

    @pl.when(sid < NS - 1)
    def _():
        pltpu.sync_copy(acc.at[pl.ds(row0, ROWS_T)],
                        out_hbm.at[pl.ds(base + row0, ROWS_T)])

    @pl.when(sid == NS - 1)
    def _():
        pltpu.sync_copy(acc.at[pl.ds(row0, ROWS_LAST)],
                        out_hbm.at[pl.ds(base + row0, ROWS_LAST)])


_RB = 1000


def _mm1_body(x_ref, w_ref, deg_ref, out_ref):
    dinv = lax.rsqrt(deg_ref[...])
    p = jnp.dot(x_ref[...], w_ref[...], preferred_element_type=jnp.float32)
    out_ref[0] = dinv * p


def _tc_mm1(x, W1, deg2d):
    return pl.pallas_call(
        _mm1_body,
        grid=(N // _RB, NC),
        in_specs=[
            pl.BlockSpec((_RB, F_IN), lambda r, c: (r, 0)),
            pl.BlockSpec((F_IN, HALF), lambda r, c: (0, c)),
            pl.BlockSpec((_RB, 1), lambda r, c: (r, 0)),
        ],
        out_specs=pl.BlockSpec((1, _RB, HALF), lambda r, c: (c, r, 0)),
        out_shape=jax.ShapeDtypeStruct((NC, N, HALF), jnp.float32),
    )(x, W1, deg2d)


def _layer_body(sh_ref, deg_ref, b_ref, w_ref, out_ref):
    dinv = lax.rsqrt(deg_ref[...])
    a0 = jnp.maximum(dinv * sh_ref[0] + b_ref[0], 0.0)
    a1 = jnp.maximum(dinv * sh_ref[1] + b_ref[1], 0.0)
    xcat = jnp.concatenate([a0, a1], axis=1)
    p = jnp.dot(xcat, w_ref[...], preferred_element_type=jnp.float32)
    out_ref[0] = dinv * p


def _tc_layer(sh, deg2d, b2d, W):
    return pl.pallas_call(
        _layer_body,
        grid=(N // _RB, NC),
        in_specs=[
            pl.BlockSpec((NC, _RB, HALF), lambda r, c: (0, r, 0)),
            pl.BlockSpec((_RB, 1), lambda r, c: (r, 0)),
            pl.BlockSpec((NC, HALF), lambda r, c: (0, 0)),
            pl.BlockSpec((F, HALF), lambda r, c: (0, c)),
        ],
        out_specs=pl.BlockSpec((1, _RB, HALF), lambda r, c: (c, r, 0)),
        out_shape=jax.ShapeDtypeStruct((NC, N, HALF), jnp.float32),
    )(sh, deg2d, b2d, W)


def _final_body(sh_ref, deg_ref, b_ref, out_ref):
    dinv = lax.rsqrt(deg_ref[...])
    h = jnp.concatenate([sh_ref[0], sh_ref[1]], axis=1)
    out_ref[...] = dinv * h + b_ref[...]


def _tc_final(sh, deg2d, b1row):
    return pl.pallas_call(
        _final_body,
        grid=(N // _RB,),
        in_specs=[
            pl.BlockSpec((NC, _RB, HALF), lambda r: (0, r, 0)),
            pl.BlockSpec((_RB, 1), lambda r: (r, 0)),
            pl.BlockSpec((1, F), lambda r: (0, 0)),
        ],
        out_specs=pl.BlockSpec((_RB, F), lambda r: (r, 0)),
        out_shape=jax.ShapeDtypeStruct((N, F), jnp.float32),
    )(sh, deg2d, b1row)


def kernel(x, edge_index, W1, b1, W2, b2, W3, b3):
    src = edge_index[0].astype(jnp.int32)
    dst = edge_index[1].astype(jnp.int32)
    pad = jnp.arange(E_P - E, dtype=jnp.int32)
    src1p = jnp.concatenate([src, pad % N])
    dst1p = jnp.concatenate([dst, N + pad % ACC_PAD])
    dst2d = dst1p.reshape(E_P // CHUNK, CHUNK)
    ones_stage = jnp.ones((ROWS_T,), jnp.float32)

    deg = _sc_degree(dst1p, ones_stage)[:N]
    deg2d = deg[:, None]

    hp1 = _tc_mm1(x, W1, deg2d)
    sh1 = _sc_aggregate(hp1.reshape(NC * N, HALF), src1p, dst2d)
    hp2 = _tc_layer(sh1.reshape(NC, N, HALF), deg2d, b1.reshape(NC, HALF), W2)
    sh2 = _sc_aggregate(hp2.reshape(NC * N, HALF), src1p, dst2d)
    hp3 = _tc_layer(sh2.reshape(NC, N, HALF), deg2d, b2.reshape(NC, HALF), W3)
    sh3 = _sc_aggregate(hp3.reshape(NC * N, HALF), src1p, dst2d)
    return _tc_final(sh3.reshape(NC, N, HALF), deg2d, b3.reshape(1, F))


# speedup vs baseline: 1.1765x; 1.1765x over previous
import functools

import jax
import jax.numpy as jnp
from jax import lax
from jax.experimental import pallas as pl
from jax.experimental.pallas import tpu as pltpu
from jax.experimental.pallas import tpu_sc as plsc

N = 10000
E = 320000
F_IN = 128
F = 256
HALF = 128
NC = 2
NS = 16
CHUNK = 128
EPT = E // NS
NFULL = EPT // CHUNK
REM = EPT - NFULL * CHUNK
ROWS_T = 640
ROWS_LAST = N - (NS - 1) * ROWS_T
DEGW = 16

@functools.cache
def _mesh():
    return plsc.VectorSubcoreMesh(core_axis_name="c", subcore_axis_name="s",
                                  num_cores=NC, num_subcores=NS)


NP = NS * ROWS_T


def _sc_degree(dst1p, ones_stage):
    return pl.kernel(
        _sc_degree_body,
        out_type=jax.ShapeDtypeStruct((NP,), jnp.float32),
        mesh=_mesh(),
        scratch_types=[
            pltpu.VMEM_SHARED((NP,), jnp.float32),
            pltpu.VMEM((ROWS_T,), jnp.float32),
            pltpu.VMEM((CHUNK,), jnp.float32),
            pltpu.VMEM((CHUNK,), jnp.int32),
            pltpu.VMEM((CHUNK,), jnp.int32),
            pltpu.VMEM((CHUNK,), jnp.int32),
            pltpu.VMEM((CHUNK,), jnp.int32),
            pltpu.SemaphoreType.DMA,
            pltpu.SemaphoreType.DMA,
            pltpu.SemaphoreType.DMA,
            pltpu.SemaphoreType.DMA,
        ],
    )(dst1p, ones_stage)


def _sc_degree_body(dst_hbm, ones_hbm, out_hbm, acc, ones_t, ones_c,
                    dst0, dst1, dst2, dst3, s0, s1, s2, s3):
    cid = lax.axis_index("c")
    sid = lax.axis_index("s")
    row0 = sid * ROWS_T
    D = (dst0, dst1, dst2, dst3)
    SEM = (s0, s1, s2, s3)

    pltpu.sync_copy(ones_hbm, ones_t)
    pltpu.sync_copy(ones_hbm.at[pl.ds(0, CHUNK)], ones_c)
    pltpu.sync_copy(ones_t, acc.at[pl.ds(row0, ROWS_T)])

    plsc.subcore_barrier()

    ebase = sid * (ERT * CHUNK)

    def loadD(k, q):
        pltpu.async_copy(dst_hbm.at[pl.ds(ebase + k * CHUNK, CHUNK)],
                         D[q], SEM[q])

    def waitD(q):
        pltpu.make_async_copy(dst_hbm.at[pl.ds(0, CHUNK)],
                              D[q], SEM[q]).wait()

    for q in range(4):
        loadD(q, q)

    def body(p, c):
        for i in range(4):
            waitD(i)
            pltpu.sync_copy(ones_c, acc.at[D[i]], add=True)
            loadD(4 * p + i + 4, i)
        return c

    lax.fori_loop(0, ERT // 4 - 1, body, 0)
    for i in range(4):
        waitD(i)
        pltpu.sync_copy(ones_c, acc.at[D[i]], add=True)

    plsc.subcore_barrier()

    @pl.when(cid == 0)
    def _():
        pltpu.sync_copy(acc.at[pl.ds(row0, ROWS_T)],
                        out_hbm.at[pl.ds(row0, ROWS_T)])


ERT = 160
E_P = ERT * NS * CHUNK
ACC_PAD = 64
ACC_R = N + ACC_PAD


BLK = 8
NBLK = ERT // BLK


def _sc_aggregate(hp, src1p, dst2d):
    return pl.kernel(
        _sc_aggregate_body,
        out_type=jax.ShapeDtypeStruct((NC * N, HALF), jnp.float32),
        mesh=_mesh(),
        scratch_types=[
            pltpu.VMEM_SHARED((ACC_R, HALF), jnp.float32),
            pltpu.VMEM((BLK * CHUNK,), jnp.int32),
            pltpu.VMEM((BLK * CHUNK,), jnp.int32),
            pltpu.VMEM((BLK, CHUNK), jnp.int32),
            pltpu.VMEM((BLK, CHUNK), jnp.int32),
            pltpu.VMEM((CHUNK, HALF), jnp.float32),
            pltpu.VMEM((CHUNK, HALF), jnp.float32),
            pltpu.SemaphoreType.DMA,
            pltpu.SemaphoreType.DMA,
            pltpu.SemaphoreType.DMA,
            pltpu.SemaphoreType.DMA,
        ],
    )(hp, src1p, dst2d)


def _sc_aggregate_body(hp_hbm, src_hbm, dst_hbm, out_hbm, acc,
                       srcb0, srcb1, dstb0, dstb1, rowsA, rowsB,
                       sb0, sb1, sgA, sgB):
    cid = lax.axis_index("c")
    sid = lax.axis_index("s")
    base = cid * N
    row0 = sid * ROWS_T
    SRCB = (srcb0, srcb1)
    DSTB = (dstb0, dstb1)
    SB = (sb0, sb1)
    ROWS = (rowsA, rowsB)
    SG = (sgA, sgB)

    @pl.when(sid < NS - 1)
    def _():
        pltpu.sync_copy(hp_hbm.at[pl.ds(base + row0, ROWS_T)],
                        acc.at[pl.ds(row0, ROWS_T)])

    @pl.when(sid == NS - 1)
    def _():
        pltpu.sync_copy(hp_hbm.at[pl.ds(base + row0, ROWS_LAST)],
                        acc.at[pl.ds(row0, ROWS_LAST)])

    plsc.subcore_barrier()

    ebase = sid * (ERT * CHUNK)
    brow = sid * ERT

    def loadblk(b, q):
        pltpu.async_copy(src_hbm.at[pl.ds(ebase + b * (BLK * CHUNK),
                                          BLK * CHUNK)], SRCB[q], SB[q])
        pltpu.async_copy(dst_hbm.at[pl.ds(brow + b * BLK, BLK)],
                         DSTB[q], SB[q])

    def waitblk(q):
        pltpu.make_async_copy(src_hbm.at[pl.ds(0, BLK * CHUNK)],
                              SRCB[q], SB[q]).wait()
        pltpu.make_async_copy(dst_hbm.at[pl.ds(0, BLK)],
                              DSTB[q], SB[q]).wait()

    def addbase(q):
        for j in range(BLK * CHUNK // 16):
            sl = pl.ds(j * 16, 16)
            SRCB[q][sl] = SRCB[q][sl] + base

    def start_gather(q, i, r):
        pltpu.async_copy(hp_hbm.at[SRCB[q].at[pl.ds(i * CHUNK, CHUNK)]],
                         ROWS[r], SG[r])

    def wait_gather(r):
        pltpu.make_async_copy(hp_hbm.at[SRCB[0].at[pl.ds(0, CHUNK)]],
                              ROWS[r], SG[r]).wait()

    def scatter(q, i, r):
        pltpu.sync_copy(ROWS[r], acc.at[DSTB[q].at[i]], add=True)

    loadblk(0, 0)
    loadblk(1, 1)
    waitblk(0)
    addbase(0)
    start_gather(0, 0, 0)
    start_gather(0, 1, 1)

    def halfblock(q, nextq, pre_seam=None):
        for i in range(BLK):
            r = i % 2
            wait_gather(r)
            scatter(q, i, r)
            if i == BLK - 2 and pre_seam is not None:
                pre_seam()
            if i + 2 < BLK:
                start_gather(q, i + 2, r)
            elif nextq is not None:
                start_gather(nextq, i + 2 - BLK, r)

    def ready(q):
        def f():
            waitblk(q)
            addbase(q)
        return f

    def body(p, c):
        b0 = 2 * p
        halfblock(0, 1, pre_seam=ready(1))
        loadblk(b0 + 2, 0)
        halfblock(1, 0, pre_seam=ready(0))
        loadblk(b0 + 3, 1)
        return c

    lax.fori_loop(0, NBLK // 2 - 1, body, 0)
    halfblock(0, 1, pre_seam=ready(1))
    halfblock(1, None)

    plsc.subcore_barrier()

    @pl.when(sid < NS - 1)
    def _():
        pltpu.sync_copy(acc.at[pl.ds(row0, ROWS_T)],
                        out_hbm.at[pl.ds(base + row0, ROWS_T)])

    @pl.when(sid == NS - 1)
    def _():
        pltpu.sync_copy(acc.at[pl.ds(row0, ROWS_LAST)],
                        out_hbm.at[pl.ds(base + row0, ROWS_LAST)])


_RB = 1000


def _mm1_body(x_ref, w_ref, deg_ref, out_ref):
    dinv = lax.rsqrt(deg_ref[...])
    p = jnp.dot(x_ref[...], w_ref[...], preferred_element_type=jnp.float32)
    out_ref[0] = dinv * p


def _tc_mm1(x, W1, deg2d):
    return pl.pallas_call(
        _mm1_body,
        grid=(N // _RB, NC),
        in_specs=[
            pl.BlockSpec((_RB, F_IN), lambda r, c: (r, 0)),
            pl.BlockSpec((F_IN, HALF), lambda r, c: (0, c)),
            pl.BlockSpec((_RB, 1), lambda r, c: (r, 0)),
        ],
        out_specs=pl.BlockSpec((1, _RB, HALF), lambda r, c: (c, r, 0)),
        out_shape=jax.ShapeDtypeStruct((NC, N, HALF), jnp.float32),
    )(x, W1, deg2d)


def _layer_body(sh_ref, deg_ref, b_ref, w_ref, out_ref):
    dinv = lax.rsqrt(deg_ref[...])
    a0 = jnp.maximum(dinv * sh_ref[0] + b_ref[0], 0.0)
    a1 = jnp.maximum(dinv * sh_ref[1] + b_ref[1], 0.0)
    xcat = jnp.concatenate([a0, a1], axis=1)
    p = jnp.dot(xcat, w_ref[...], preferred_element_type=jnp.float32)
    out_ref[0] = dinv * p


def _tc_layer(sh, deg2d, b2d, W):
    return pl.pallas_call(
        _layer_body,
        grid=(N // _RB, NC),
        in_specs=[
            pl.BlockSpec((NC, _RB, HALF), lambda r, c: (0, r, 0)),
            pl.BlockSpec((_RB, 1), lambda r, c: (r, 0)),
            pl.BlockSpec((NC, HALF), lambda r, c: (0, 0)),
            pl.BlockSpec((F, HALF), lambda r, c: (0, c)),
        ],
        out_specs=pl.BlockSpec((1, _RB, HALF), lambda r, c: (c, r, 0)),
        out_shape=jax.ShapeDtypeStruct((NC, N, HALF), jnp.float32),
    )(sh, deg2d, b2d, W)


def _final_body(sh_ref, deg_ref, b_ref, out_ref):
    dinv = lax.rsqrt(deg_ref[...])
    h = jnp.concatenate([sh_ref[0], sh_ref[1]], axis=1)
    out_ref[...] = dinv * h + b_ref[...]


def _tc_final(sh, deg2d, b1row):
    return pl.pallas_call(
        _final_body,
        grid=(N // _RB,),
        in_specs=[
            pl.BlockSpec((NC, _RB, HALF), lambda r: (0, r, 0)),
            pl.BlockSpec((_RB, 1), lambda r: (r, 0)),
            pl.BlockSpec((1, F), lambda r: (0, 0)),
        ],
        out_specs=pl.BlockSpec((_RB, F), lambda r: (r, 0)),
        out_shape=jax.ShapeDtypeStruct((N, F), jnp.float32),
    )(sh, deg2d, b1row)


def kernel(x, edge_index, W1, b1, W2, b2, W3, b3):
    src = edge_index[0].astype(jnp.int32)
    dst = edge_index[1].astype(jnp.int32)
    pad = jnp.arange(E_P - E, dtype=jnp.int32)
    src1p = jnp.concatenate([src, pad % N])
    dst1p = jnp.concatenate([dst, N + pad % ACC_PAD])
    dst2d = dst1p.reshape(E_P // CHUNK, CHUNK)
    ones_stage = jnp.ones((ROWS_T,), jnp.float32)

    deg = _sc_degree(dst1p, ones_stage)[:N]
    deg2d = deg[:, None]

    hp1 = _tc_mm1(x, W1, deg2d)
    sh1 = _sc_aggregate(hp1.reshape(NC * N, HALF), src1p, dst2d)
    hp2 = _tc_layer(sh1.reshape(NC, N, HALF), deg2d, b1.reshape(NC, HALF), W2)
    sh2 = _sc_aggregate(hp2.reshape(NC * N, HALF), src1p, dst2d)
    hp3 = _tc_layer(sh2.reshape(NC, N, HALF), deg2d, b2.reshape(NC, HALF), W3)
    sh3 = _sc_aggregate(hp3.reshape(NC * N, HALF), src1p, dst2d)
    return _tc_final(sh3.reshape(NC, N, HALF), deg2d, b3.reshape(1, F))


# speedup vs baseline: 1.1789x; 1.0020x over previous
import functools

import jax
import jax.numpy as jnp
from jax import lax
from jax.experimental import pallas as pl
from jax.experimental.pallas import tpu as pltpu
from jax.experimental.pallas import tpu_sc as plsc

N = 10000
E = 320000
F_IN = 128
F = 256
HALF = 128
NC = 2
NS = 16
CHUNK = 128
ROWS_T = 640
ROWS_LAST = N - (NS - 1) * ROWS_T

@functools.cache
def _mesh():
    return plsc.VectorSubcoreMesh(core_axis_name="c", subcore_axis_name="s",
                                  num_cores=NC, num_subcores=NS)


NP = NS * ROWS_T


def _sc_degree(dst1p, ones_stage):
    return pl.kernel(
        _sc_degree_body,
        out_type=jax.ShapeDtypeStruct((NP,), jnp.float32),
        mesh=_mesh(),
        scratch_types=[
            pltpu.VMEM_SHARED((NP,), jnp.float32),
            pltpu.VMEM((ROWS_T,), jnp.float32),
            pltpu.VMEM((CHUNK,), jnp.float32),
            pltpu.VMEM((CHUNK,), jnp.int32),
            pltpu.VMEM((CHUNK,), jnp.int32),
            pltpu.VMEM((CHUNK,), jnp.int32),
            pltpu.VMEM((CHUNK,), jnp.int32),
            pltpu.SemaphoreType.DMA,
            pltpu.SemaphoreType.DMA,
            pltpu.SemaphoreType.DMA,
            pltpu.SemaphoreType.DMA,
        ],
    )(dst1p, ones_stage)


def _sc_degree_body(dst_hbm, ones_hbm, out_hbm, acc, ones_t, ones_c,
                    dst0, dst1, dst2, dst3, s0, s1, s2, s3):
    cid = lax.axis_index("c")
    sid = lax.axis_index("s")
    row0 = sid * ROWS_T
    D = (dst0, dst1, dst2, dst3)
    SEM = (s0, s1, s2, s3)

    pltpu.sync_copy(ones_hbm, ones_t)
    pltpu.sync_copy(ones_hbm.at[pl.ds(0, CHUNK)], ones_c)
    pltpu.sync_copy(ones_t, acc.at[pl.ds(row0, ROWS_T)])

    plsc.subcore_barrier()

    ebase = sid * (ERT * CHUNK)

    def loadD(k, q):
        pltpu.async_copy(dst_hbm.at[pl.ds(ebase + k * CHUNK, CHUNK)],
                         D[q], SEM[q])

    def waitD(q):
        pltpu.make_async_copy(dst_hbm.at[pl.ds(0, CHUNK)],
                              D[q], SEM[q]).wait()

    for q in range(4):
        loadD(q, q)

    def body(p, c):
        for i in range(4):
            waitD(i)
            pltpu.sync_copy(ones_c, acc.at[D[i]], add=True)
            loadD(4 * p + i + 4, i)
        return c

    lax.fori_loop(0, ERT // 4 - 1, body, 0)
    for i in range(4):
        waitD(i)
        pltpu.sync_copy(ones_c, acc.at[D[i]], add=True)

    plsc.subcore_barrier()

    @pl.when(cid == 0)
    def _():
        pltpu.sync_copy(acc.at[pl.ds(row0, ROWS_T)],
                        out_hbm.at[pl.ds(row0, ROWS_T)])


ERT = 160
E_P = ERT * NS * CHUNK
ACC_PAD = 64
ACC_R = N + ACC_PAD


BLK = 8
NBLK = ERT // BLK


def _sc_aggregate(hp, src1p, dst2d):
    return pl.kernel(
        _sc_aggregate_body,
        out_type=jax.ShapeDtypeStruct((NC * N, HALF), jnp.float32),
        mesh=_mesh(),
        scratch_types=[
            pltpu.VMEM_SHARED((ACC_R, HALF), jnp.float32),
            pltpu.VMEM((BLK * CHUNK,), jnp.int32),
            pltpu.VMEM((BLK * CHUNK,), jnp.int32),
            pltpu.VMEM((BLK, CHUNK), jnp.int32),
            pltpu.VMEM((BLK, CHUNK), jnp.int32),
            pltpu.VMEM((CHUNK, HALF), jnp.float32),
            pltpu.VMEM((CHUNK, HALF), jnp.float32),
            pltpu.SemaphoreType.DMA,
            pltpu.SemaphoreType.DMA,
            pltpu.SemaphoreType.DMA,
            pltpu.SemaphoreType.DMA,
        ],
    )(hp, src1p, dst2d)


def _sc_aggregate_body(hp_hbm, src_hbm, dst_hbm, out_hbm, acc,
                       srcb0, srcb1, dstb0, dstb1, rowsA, rowsB,
                       sb0, sb1, sgA, sgB):
    cid = lax.axis_index("c")
    sid = lax.axis_index("s")
    base = cid * N
    row0 = sid * ROWS_T
    SRCB = (srcb0, srcb1)
    DSTB = (dstb0, dstb1)
    SB = (sb0, sb1)
    ROWS = (rowsA, rowsB)
    SG = (sgA, sgB)

    @pl.when(sid < NS - 1)
    def _():
        pltpu.sync_copy(hp_hbm.at[pl.ds(base + row0, ROWS_T)],
                        acc.at[pl.ds(row0, ROWS_T)])

    @pl.when(sid == NS - 1)
    def _():
        pltpu.sync_copy(hp_hbm.at[pl.ds(base + row0, ROWS_LAST)],
                        acc.at[pl.ds(row0, ROWS_LAST)])

    plsc.subcore_barrier()

    ebase = sid * (ERT * CHUNK)
    brow = sid * ERT

    def loadblk(b, q):
        pltpu.async_copy(src_hbm.at[pl.ds(ebase + b * (BLK * CHUNK),
                                          BLK * CHUNK)], SRCB[q], SB[q])
        pltpu.async_copy(dst_hbm.at[pl.ds(brow + b * BLK, BLK)],
                         DSTB[q], SB[q])

    def waitblk(q):
        pltpu.make_async_copy(src_hbm.at[pl.ds(0, BLK * CHUNK)],
                              SRCB[q], SB[q]).wait()
        pltpu.make_async_copy(dst_hbm.at[pl.ds(0, BLK)],
                              DSTB[q], SB[q]).wait()

    def addbase(q):
        for j in range(BLK * CHUNK // 16):
            sl = pl.ds(j * 16, 16)
            SRCB[q][sl] = SRCB[q][sl] + base

    def start_gather(q, i, r):
        pltpu.async_copy(hp_hbm.at[SRCB[q].at[pl.ds(i * CHUNK, CHUNK)]],
                         ROWS[r], SG[r])

    def wait_gather(r):
        pltpu.make_async_copy(hp_hbm.at[SRCB[0].at[pl.ds(0, CHUNK)]],
                              ROWS[r], SG[r]).wait()

    def scatter(q, i, r):
        pltpu.sync_copy(ROWS[r], acc.at[DSTB[q].at[i]], add=True)

    loadblk(0, 0)
    loadblk(1, 1)
    waitblk(0)
    addbase(0)
    start_gather(0, 0, 0)
    start_gather(0, 1, 1)

    def halfblock(q, nextq, pre_seam=None):
        for i in range(BLK):
            r = i % 2
            wait_gather(r)
            scatter(q, i, r)
            if i == BLK - 2 and pre_seam is not None:
                pre_seam()
            if i + 2 < BLK:
                start_gather(q, i + 2, r)
            elif nextq is not None:
                start_gather(nextq, i + 2 - BLK, r)

    def ready(q):
        def f():
            waitblk(q)
            addbase(q)
        return f

    def body(p, c):
        b0 = 2 * p
        halfblock(0, 1, pre_seam=ready(1))
        loadblk(b0 + 2, 0)
        halfblock(1, 0, pre_seam=ready(0))
        loadblk(b0 + 3, 1)
        return c

    lax.fori_loop(0, NBLK // 2 - 1, body, 0)
    halfblock(0, 1, pre_seam=ready(1))
    halfblock(1, None)

    plsc.subcore_barrier()

    @pl.when(sid < NS - 1)
    def _():
        pltpu.sync_copy(acc.at[pl.ds(row0, ROWS_T)],
                        out_hbm.at[pl.ds(base + row0, ROWS_T)])

    @pl.when(sid == NS - 1)
    def _():
        pltpu.sync_copy(acc.at[pl.ds(row0, ROWS_LAST)],
                        out_hbm.at[pl.ds(base + row0, ROWS_LAST)])


_RB = 1000


def _mm1_body(x_ref, w_ref, deg_ref, out_ref):
    dinv = lax.rsqrt(deg_ref[...])
    p = jnp.dot(x_ref[...], w_ref[...], preferred_element_type=jnp.float32)
    out_ref[0] = dinv * p


def _tc_mm1(x, W1, deg2d):
    return pl.pallas_call(
        _mm1_body,
        grid=(N // _RB, NC),
        in_specs=[
            pl.BlockSpec((_RB, F_IN), lambda r, c: (r, 0)),
            pl.BlockSpec((F_IN, HALF), lambda r, c: (0, c)),
            pl.BlockSpec((_RB, 1), lambda r, c: (r, 0)),
        ],
        out_specs=pl.BlockSpec((1, _RB, HALF), lambda r, c: (c, r, 0)),
        out_shape=jax.ShapeDtypeStruct((NC, N, HALF), jnp.float32),
    )(x, W1, deg2d)


def _layer_body(sh_ref, deg_ref, b_ref, w_ref, out_ref):
    dinv = lax.rsqrt(deg_ref[...])
    a0 = jnp.maximum(dinv * sh_ref[0] + b_ref[0], 0.0)
    a1 = jnp.maximum(dinv * sh_ref[1] + b_ref[1], 0.0)
    xcat = jnp.concatenate([a0, a1], axis=1)
    p = jnp.dot(xcat, w_ref[...], preferred_element_type=jnp.float32)
    out_ref[0] = dinv * p


def _tc_layer(sh, deg2d, b2d, W):
    return pl.pallas_call(
        _layer_body,
        grid=(N // _RB, NC),
        in_specs=[
            pl.BlockSpec((NC, _RB, HALF), lambda r, c: (0, r, 0)),
            pl.BlockSpec((_RB, 1), lambda r, c: (r, 0)),
            pl.BlockSpec((NC, HALF), lambda r, c: (0, 0)),
            pl.BlockSpec((F, HALF), lambda r, c: (0, c)),
        ],
        out_specs=pl.BlockSpec((1, _RB, HALF), lambda r, c: (c, r, 0)),
        out_shape=jax.ShapeDtypeStruct((NC, N, HALF), jnp.float32),
    )(sh, deg2d, b2d, W)


def _final_body(sh_ref, deg_ref, b_ref, out_ref):
    dinv = lax.rsqrt(deg_ref[...])
    h = jnp.concatenate([sh_ref[0], sh_ref[1]], axis=1)
    out_ref[...] = dinv * h + b_ref[...]


def _tc_final(sh, deg2d, b1row):
    return pl.pallas_call(
        _final_body,
        grid=(N // _RB,),
        in_specs=[
            pl.BlockSpec((NC, _RB, HALF), lambda r: (0, r, 0)),
            pl.BlockSpec((_RB, 1), lambda r: (r, 0)),
            pl.BlockSpec((1, F), lambda r: (0, 0)),
        ],
        out_specs=pl.BlockSpec((_RB, F), lambda r: (r, 0)),
        out_shape=jax.ShapeDtypeStruct((N, F), jnp.float32),
    )(sh, deg2d, b1row)


def kernel(x, edge_index, W1, b1, W2, b2, W3, b3):
    src = edge_index[0].astype(jnp.int32)
    dst = edge_index[1].astype(jnp.int32)
    pad = jnp.arange(E_P - E, dtype=jnp.int32)
    src1p = jnp.concatenate([src, pad % N])
    dst1p = jnp.concatenate([dst, N + pad % ACC_PAD])
    dst2d = dst1p.reshape(E_P // CHUNK, CHUNK)
    ones_stage = jnp.ones((ROWS_T,), jnp.float32)

    deg = _sc_degree(dst1p, ones_stage)[:N]
    deg2d = deg[:, None]

    hp1 = _tc_mm1(x, W1, deg2d)
    sh1 = _sc_aggregate(hp1.reshape(NC * N, HALF), src1p, dst2d)
    hp2 = _tc_layer(sh1.reshape(NC, N, HALF), deg2d, b1.reshape(NC, HALF), W2)
    sh2 = _sc_aggregate(hp2.reshape(NC * N, HALF), src1p, dst2d)
    hp3 = _tc_layer(sh2.reshape(NC, N, HALF), deg2d, b2.reshape(NC, HALF), W3)
    sh3 = _sc_aggregate(hp3.reshape(NC * N, HALF), src1p, dst2d)
    return _tc_final(sh3.reshape(NC, N, HALF), deg2d, b3.reshape(1, F))
